# Initial kernel scaffold; baseline (speedup 1.0000x reference)
import functools

import jax
import jax.numpy as jnp
from jax import lax
from jax.experimental import pallas as pl
from jax.experimental.pallas import tpu as pltpu
from jax.experimental.pallas import tpu_sc as plsc

N = 20000
NN = 10000
E = 320000
F = 128
C = 8
NCLS = 10
SLOPE = 0.01

NC = 2
NS = 16
NW = NC * NS
K = 128
NCH = (E + NW * K - 1) // (NW * K)
EPAD = NW * K * NCH
NPAD = 20096
SL = NPAD // NS

_mesh = plsc.VectorSubcoreMesh(core_axis_name="c", subcore_axis_name="s")


@functools.partial(
    pl.kernel,
    out_type=jax.ShapeDtypeStruct((NC, NPAD), jnp.float32),
    mesh=_mesh,
    scratch_types=[
        pltpu.VMEM((NCH, K), jnp.int32),
        pltpu.VMEM((K,), jnp.float32),
        pltpu.VMEM((SL,), jnp.float32),
        pltpu.VMEM_SHARED((NPAD,), jnp.float32),
    ],
)
def _deg_kernel(col_hbm, zeros1_hbm, deg_out, colv, onesv, stage, acc):
    cid = lax.axis_index("c")
    sid = lax.axis_index("s")
    wid = cid * NS + sid
    for i in range(K // 16):
        onesv[pl.ds(i * 16, 16)] = jnp.ones((16,), jnp.float32)
    pltpu.sync_copy(zeros1_hbm.at[pl.ds(sid * SL, SL)], stage)
    pltpu.sync_copy(stage, acc.at[pl.ds(sid * SL, SL)])
    pltpu.sync_copy(col_hbm.at[wid], colv)
    plsc.subcore_barrier()

    def body(j, carry):
        pltpu.sync_copy(onesv, acc.at[colv.at[j]], add=True)
        return carry

    lax.fori_loop(0, NCH, body, 0)
    plsc.subcore_barrier()
    pltpu.sync_copy(acc.at[pl.ds(sid * SL, SL)], stage)
    pltpu.sync_copy(stage, deg_out.at[cid, pl.ds(sid * SL, SL)])


@functools.partial(
    pl.kernel,
    out_type=jax.ShapeDtypeStruct((NC, NPAD, C), jnp.float32),
    mesh=_mesh,
    scratch_types=[
        pltpu.VMEM((NCH, K), jnp.int32),
        pltpu.VMEM((NCH, K), jnp.int32),
        pltpu.VMEM((K, C), jnp.float32),
        pltpu.VMEM((SL, C), jnp.float32),
        pltpu.VMEM_SHARED((NPAD, C), jnp.float32),
    ],
)
def _scatter_kernel(g_hbm, row_hbm, col_hbm, zeros8_hbm, s_out,
                    rowv, colv, rbuf, stage, acc):
    cid = lax.axis_index("c")
    sid = lax.axis_index("s")
    wid = cid * NS + sid
    pltpu.sync_copy(zeros8_hbm.at[pl.ds(sid * SL, SL)], stage)
    pltpu.sync_copy(stage, acc.at[pl.ds(sid * SL, SL)])
    pltpu.sync_copy(row_hbm.at[wid], rowv)
    pltpu.sync_copy(col_hbm.at[wid], colv)
    plsc.subcore_barrier()

    def body(j, carry):
        pltpu.sync_copy(g_hbm.at[rowv.at[j]], rbuf)
        pltpu.sync_copy(rbuf, acc.at[colv.at[j]], add=True)
        return carry

    lax.fori_loop(0, NCH, body, 0)
    plsc.subcore_barrier()
    pltpu.sync_copy(acc.at[pl.ds(sid * SL, SL)], stage)
    pltpu.sync_copy(stage, s_out.at[cid, pl.ds(sid * SL, SL)])


def _g_body(x_ref, w_ref, degp_ref, g_ref):
    h = jnp.dot(x_ref[...], w_ref[...], preferred_element_type=jnp.float32)
    deg = degp_ref[0] + degp_ref[1] + 1.0
    dinv = lax.rsqrt(deg)
    g = h * dinv[:N]
    g_ref[...] = jnp.concatenate(
        [g, jnp.zeros((NPAD - N, C), jnp.float32)], axis=0)


_g_call = pl.pallas_call(
    _g_body,
    out_shape=jax.ShapeDtypeStruct((NPAD, C), jnp.float32),
)


def _leaky(v):
    return jnp.where(v >= 0, v, SLOPE * v)


def _head_body(sp_ref, g_ref, degp_ref, bg_ref, w1_ref, b1_ref, w2_ref,
               b2_ref, out_ref):
    deg = degp_ref[0] + degp_ref[1] + 1.0
    dinv = lax.rsqrt(deg)
    s = (sp_ref[0] + sp_ref[1] + g_ref[...]) * dinv + bg_ref[...]
    a = jnp.dot(_leaky(s), w1_ref[...],
                preferred_element_type=jnp.float32) + b1_ref[0, 0]
    y = _leaky(a)
    w2 = w2_ref[...]
    z0 = jnp.sum(y[0:NN] * w2, axis=0, keepdims=True)
    z1 = jnp.sum(y[NN:2 * NN] * w2, axis=0, keepdims=True)
    z = jnp.concatenate([z0, z1], axis=0) + b2_ref[...]
    m = jnp.max(z, axis=1, keepdims=True)
    lse = jnp.log(jnp.sum(jnp.exp(z - m), axis=1, keepdims=True)) + m
    out_ref[...] = z - lse


_head_call = pl.pallas_call(
    _head_body,
    out_shape=jax.ShapeDtypeStruct((2, NCLS), jnp.float32),
)


def kernel(x, edge_index, batch, W_gcn, b_gcn, W_fc1, b_fc1, W_fc2, b_fc2):
    del batch
    pad = EPAD - E
    row = jnp.concatenate([edge_index[0], jnp.full((pad,), N, jnp.int32)])
    col = jnp.concatenate([edge_index[1], jnp.full((pad,), N, jnp.int32)])
    rowp = row.reshape(NW, NCH, K)
    colp = col.reshape(NW, NCH, K)
    zeros1 = jnp.zeros((NPAD,), jnp.float32)
    zeros8 = jnp.zeros((NPAD, C), jnp.float32)

    degp = _deg_kernel(colp, zeros1)
    degp3 = degp.reshape(NC, NPAD, 1)
    g = _g_call(x, W_gcn, degp3)
    sp = _scatter_kernel(g, rowp, colp, zeros8)
    out = _head_call(sp, g, degp3, b_gcn.reshape(1, C), W_fc1,
                     b_fc1.reshape(1, 1), W_fc2, b_fc2.reshape(1, NCLS))
    return out


# speedup vs baseline: 40.5339x; 40.5339x over previous
import functools

import jax
import jax.numpy as jnp
from jax import lax
from jax.experimental import pallas as pl
from jax.experimental.pallas import tpu as pltpu
from jax.experimental.pallas import tpu_sc as plsc

N = 20000
NN = 10000
E = 320000
F = 128
C = 8
NCLS = 10
SLOPE = 0.01

NC = 2
NS = 16
NW = NC * NS
K = 128
NCH = (E + NW * K - 1) // (NW * K)
EPAD = NW * K * NCH
NPAD = 20096
SL = NPAD // NS

def _deg_body(col_hbm, zeros1_hbm, deg_out, colv, onesv, stage, acc):
    cid = lax.axis_index("c")
    sid = lax.axis_index("s")
    wid = cid * NS + sid
    for i in range(K // 16):
        onesv[pl.ds(i * 16, 16)] = jnp.ones((16,), jnp.float32)
    pltpu.sync_copy(zeros1_hbm.at[pl.ds(sid * SL, SL)], stage)
    pltpu.sync_copy(stage, acc.at[pl.ds(sid * SL, SL)])
    pltpu.sync_copy(col_hbm.at[wid], colv)
    plsc.subcore_barrier()

    def body(j, carry):
        pltpu.sync_copy(onesv, acc.at[colv.at[j]], add=True)
        return carry

    lax.fori_loop(0, NCH, body, 0)
    plsc.subcore_barrier()
    pltpu.sync_copy(acc.at[pl.ds(sid * SL, SL)], stage)
    pltpu.sync_copy(stage, deg_out.at[pl.ds(cid * NPAD + sid * SL, SL)])


def _scatter_body(g_hbm, row_hbm, col_hbm, zeros8_hbm, s_out,
                  rowv, colv, rbuf, stage, g_sh, acc):
    cid = lax.axis_index("c")
    sid = lax.axis_index("s")
    wid = cid * NS + sid
    pltpu.sync_copy(zeros8_hbm.at[pl.ds(sid * SL, SL)], stage)
    pltpu.sync_copy(stage, acc.at[pl.ds(sid * SL, SL)])
    pltpu.sync_copy(g_hbm.at[pl.ds(sid * SL, SL)], stage)
    pltpu.sync_copy(stage, g_sh.at[pl.ds(sid * SL, SL)])
    pltpu.sync_copy(row_hbm.at[wid], rowv)
    pltpu.sync_copy(col_hbm.at[wid], colv)
    plsc.subcore_barrier()

    def body(j, carry):
        pltpu.sync_copy(g_sh.at[rowv.at[j]], rbuf)
        pltpu.sync_copy(rbuf, acc.at[colv.at[j]], add=True)
        return carry

    lax.fori_loop(0, NCH, body, 0)
    plsc.subcore_barrier()
    pltpu.sync_copy(acc.at[pl.ds(sid * SL, SL)], stage)
    pltpu.sync_copy(stage, s_out.at[pl.ds(cid * NPAD + sid * SL, SL)])


BX = 2512


def _g_body(x_ref, w_ref, degp_ref, g_ref):
    h = jnp.dot(x_ref[...], w_ref[...], preferred_element_type=jnp.float32)
    deg = degp_ref[0] + degp_ref[1] + 1.0
    g_ref[...] = h * lax.rsqrt(deg)


_g_call = pl.pallas_call(
    _g_body,
    grid=(NPAD // BX,),
    in_specs=[
        pl.BlockSpec((BX, F), lambda i: (i, 0)),
        pl.BlockSpec((F, C), lambda i: (0, 0)),
        pl.BlockSpec((NC, BX, 1), lambda i: (0, i, 0)),
    ],
    out_specs=pl.BlockSpec((BX, C), lambda i: (i, 0)),
    out_shape=jax.ShapeDtypeStruct((NPAD, C), jnp.float32),
)


BR = 2000
NB = N // BR
GB = NN // BR


def _leaky(v):
    return jnp.where(v >= 0, v, SLOPE * v)


def _head_body(sp_ref, g_ref, degp_ref, bg_ref, w1_ref, b1_ref, w2_ref,
               b2_ref, out_ref, acc_ref):
    i = pl.program_id(0)

    @pl.when(i == 0)
    def _():
        acc_ref[...] = jnp.zeros_like(acc_ref)

    deg = degp_ref[0] + degp_ref[1] + 1.0
    dinv = lax.rsqrt(deg)
    s = (sp_ref[0] + sp_ref[1] + g_ref[...]) * dinv + bg_ref[...]
    a = jnp.dot(_leaky(s), w1_ref[...],
                preferred_element_type=jnp.float32) + b1_ref[0, 0]
    y = _leaky(a)
    z = jnp.sum(y * w2_ref[...], axis=0, keepdims=True)
    b = i // GB
    mask = lax.broadcasted_iota(jnp.int32, (2, 1), 0) == b
    acc_ref[...] += jnp.where(mask, z, 0.0)

    @pl.when(i == NB - 1)
    def _():
        zf = acc_ref[...] + b2_ref[...]
        m = jnp.max(zf, axis=1, keepdims=True)
        lse = jnp.log(jnp.sum(jnp.exp(zf - m), axis=1, keepdims=True)) + m
        out_ref[...] = zf - lse


_head_call = pl.pallas_call(
    _head_body,
    grid=(NB,),
    in_specs=[
        pl.BlockSpec((NC, BR, C), lambda i: (0, i, 0)),
        pl.BlockSpec((BR, C), lambda i: (i, 0)),
        pl.BlockSpec((NC, BR, 1), lambda i: (0, i, 0)),
        pl.BlockSpec((1, C), lambda i: (0, 0)),
        pl.BlockSpec((C, 1), lambda i: (0, 0)),
        pl.BlockSpec((1, 1), lambda i: (0, 0)),
        pl.BlockSpec((BR, NCLS), lambda i: (i % GB, 0)),
        pl.BlockSpec((1, NCLS), lambda i: (0, 0)),
    ],
    out_specs=pl.BlockSpec((2, NCLS), lambda i: (0, 0)),
    out_shape=jax.ShapeDtypeStruct((2, NCLS), jnp.float32),
    scratch_shapes=[pltpu.VMEM((2, NCLS), jnp.float32)],
)


@functools.cache
def _sc_kernels():
    mesh = plsc.VectorSubcoreMesh(core_axis_name="c", subcore_axis_name="s",
                                  num_cores=NC, num_subcores=NS)
    deg_kernel = pl.kernel(
        _deg_body,
        out_type=jax.ShapeDtypeStruct((NC * NPAD,), jnp.float32),
        mesh=mesh,
        compiler_params=pltpu.CompilerParams(use_tc_tiling_on_sc=False),
        scratch_types=[
            pltpu.VMEM((NCH, K), jnp.int32),
            pltpu.VMEM((K,), jnp.float32),
            pltpu.VMEM((SL,), jnp.float32),
            pltpu.VMEM_SHARED((NPAD,), jnp.float32),
        ],
    )
    scatter_kernel = pl.kernel(
        _scatter_body,
        out_type=jax.ShapeDtypeStruct((NC * NPAD, C), jnp.float32),
        mesh=mesh,
        compiler_params=pltpu.CompilerParams(use_tc_tiling_on_sc=False),
        scratch_types=[
            pltpu.VMEM((NCH, K), jnp.int32),
            pltpu.VMEM((NCH, K), jnp.int32),
            pltpu.VMEM((K, C), jnp.float32),
            pltpu.VMEM((SL, C), jnp.float32),
            pltpu.VMEM_SHARED((NPAD, C), jnp.float32),
            pltpu.VMEM_SHARED((NPAD, C), jnp.float32),
        ],
    )
    return deg_kernel, scatter_kernel


def kernel(x, edge_index, batch, W_gcn, b_gcn, W_fc1, b_fc1, W_fc2, b_fc2):
    del batch
    pad = EPAD - E
    row = jnp.concatenate([edge_index[0], jnp.full((pad,), N, jnp.int32)])
    col = jnp.concatenate([edge_index[1], jnp.full((pad,), N, jnp.int32)])
    rowp = row.reshape(NW, NCH, K)
    colp = col.reshape(NW, NCH, K)
    zeros1 = jnp.zeros((NPAD,), jnp.float32)
    zeros8 = jnp.zeros((NPAD, C), jnp.float32)

    deg_kernel, scatter_kernel = _sc_kernels()
    degp = deg_kernel(colp, zeros1)
    degp3 = degp.reshape(NC, NPAD, 1)
    xp = jnp.pad(x, ((0, NPAD - N), (0, 0)))
    g = _g_call(xp, W_gcn, degp3)
    sp = scatter_kernel(g, rowp, colp, zeros8)
    sp = sp.reshape(NC, NPAD, C)
    out = _head_call(sp, g, degp3, b_gcn.reshape(1, C), W_fc1,
                     b_fc1.reshape(1, 1), W_fc2, b_fc2.reshape(1, NCLS))
    return out


# speedup vs baseline: 45.6025x; 1.1250x over previous
import functools

import jax
import jax.numpy as jnp
from jax import lax
from jax.experimental import pallas as pl
from jax.experimental.pallas import tpu as pltpu
from jax.experimental.pallas import tpu_sc as plsc

N = 20000
NN = 10000
E = 320000
F = 128
C = 8
NCLS = 10
SLOPE = 0.01

NC = 2
NS = 16
NW = NC * NS
K = 80
NCH = E // (NW * K)
NPAD = 20096
SL = NPAD // NS

def _deg_body(col_hbm, zeros1_hbm, deg_out, colv, onesv, stage, acc):
    cid = lax.axis_index("c")
    sid = lax.axis_index("s")
    wid = cid * NS + sid
    for i in range(K // 16):
        onesv[pl.ds(i * 16, 16)] = jnp.ones((16,), jnp.float32)
    pltpu.sync_copy(zeros1_hbm.at[pl.ds(sid * SL, SL)], stage)
    pltpu.sync_copy(stage, acc.at[pl.ds(sid * SL, SL)])
    pltpu.sync_copy(col_hbm.at[wid], colv)
    plsc.subcore_barrier()

    def body(j, carry):
        pltpu.sync_copy(onesv, acc.at[colv.at[j]], add=True)
        return carry

    lax.fori_loop(0, NCH, body, 0)
    plsc.subcore_barrier()
    pltpu.sync_copy(acc.at[pl.ds(sid * SL, SL)], stage)
    pltpu.sync_copy(stage, deg_out.at[pl.ds(cid * NPAD + sid * SL, SL)])


def _scatter_body(g_hbm, row_hbm, col_hbm, zeros8_hbm, s_out,
                  rowv, colv, rbuf, stage, g_sh, acc):
    cid = lax.axis_index("c")
    sid = lax.axis_index("s")
    wid = cid * NS + sid
    pltpu.sync_copy(zeros8_hbm.at[pl.ds(sid * SL, SL)], stage)
    pltpu.sync_copy(stage, acc.at[pl.ds(sid * SL, SL)])
    pltpu.sync_copy(g_hbm.at[pl.ds(sid * SL, SL)], stage)
    pltpu.sync_copy(stage, g_sh.at[pl.ds(sid * SL, SL)])
    pltpu.sync_copy(row_hbm.at[wid], rowv)
    pltpu.sync_copy(col_hbm.at[wid], colv)
    plsc.subcore_barrier()

    def body(j, carry):
        pltpu.sync_copy(g_sh.at[rowv.at[j]], rbuf)
        pltpu.sync_copy(rbuf, acc.at[colv.at[j]], add=True)
        return carry

    lax.fori_loop(0, NCH, body, 0)
    plsc.subcore_barrier()
    pltpu.sync_copy(acc.at[pl.ds(sid * SL, SL)], stage)
    pltpu.sync_copy(stage, s_out.at[pl.ds(cid * NPAD + sid * SL, SL)])


BX = 2000


DB = 2048


def _g_body(x_ref, w_ref, d0_ref, d1_ref, g_ref):
    h = jnp.dot(x_ref[...], w_ref[...], preferred_element_type=jnp.float32)
    deg = d0_ref[...] + d1_ref[...] + 1.0
    g_ref[...] = h * jnp.reshape(lax.rsqrt(deg), (DB, 1))[:BX]


_g_call = pl.pallas_call(
    _g_body,
    grid=(N // BX,),
    in_specs=[
        pl.BlockSpec((BX, F), lambda i: (i, 0)),
        pl.BlockSpec((F, C), lambda i: (0, 0)),
        pl.BlockSpec((DB,), lambda i: (i,)),
        pl.BlockSpec((DB,), lambda i: (i,)),
    ],
    out_specs=pl.BlockSpec((BX, C), lambda i: (i, 0)),
    out_shape=jax.ShapeDtypeStruct((NPAD, C), jnp.float32),
)


BR = 2000
NB = N // BR
GB = NN // BR


def _leaky(v):
    return jnp.where(v >= 0, v, SLOPE * v)


def _head_body(sp_ref, g_ref, d0_ref, d1_ref, bg_ref, w1_ref, b1_ref, w2_ref,
               b2_ref, out_ref, acc_ref):
    i = pl.program_id(0)

    @pl.when(i == 0)
    def _():
        acc_ref[...] = jnp.zeros_like(acc_ref)

    deg = d0_ref[...] + d1_ref[...] + 1.0
    dinv = jnp.reshape(lax.rsqrt(deg), (DB, 1))[:BR]
    s = (sp_ref[0] + sp_ref[1] + g_ref[...]) * dinv + bg_ref[...]
    a = jnp.dot(_leaky(s), w1_ref[...],
                preferred_element_type=jnp.float32) + b1_ref[0, 0]
    y = _leaky(a)
    z = jnp.sum(y * w2_ref[...], axis=0, keepdims=True)
    b = i // GB
    mask = lax.broadcasted_iota(jnp.int32, (2, 1), 0) == b
    acc_ref[...] += jnp.where(mask, z, 0.0)

    @pl.when(i == NB - 1)
    def _():
        zf = acc_ref[...] + b2_ref[...]
        m = jnp.max(zf, axis=1, keepdims=True)
        lse = jnp.log(jnp.sum(jnp.exp(zf - m), axis=1, keepdims=True)) + m
        out_ref[...] = zf - lse


_head_call = pl.pallas_call(
    _head_body,
    grid=(NB,),
    in_specs=[
        pl.BlockSpec((NC, BR, C), lambda i: (0, i, 0)),
        pl.BlockSpec((BR, C), lambda i: (i, 0)),
        pl.BlockSpec((DB,), lambda i: (i,)),
        pl.BlockSpec((DB,), lambda i: (i,)),
        pl.BlockSpec((1, C), lambda i: (0, 0)),
        pl.BlockSpec((C, 1), lambda i: (0, 0)),
        pl.BlockSpec((1, 1), lambda i: (0, 0)),
        pl.BlockSpec((BR, NCLS), lambda i: (i % GB, 0)),
        pl.BlockSpec((1, NCLS), lambda i: (0, 0)),
    ],
    out_specs=pl.BlockSpec((2, NCLS), lambda i: (0, 0)),
    out_shape=jax.ShapeDtypeStruct((2, NCLS), jnp.float32),
    scratch_shapes=[pltpu.VMEM((2, NCLS), jnp.float32)],
)


@functools.cache
def _sc_kernels():
    mesh = plsc.VectorSubcoreMesh(core_axis_name="c", subcore_axis_name="s",
                                  num_cores=NC, num_subcores=NS)
    deg_kernel = pl.kernel(
        _deg_body,
        out_type=jax.ShapeDtypeStruct((NC * NPAD,), jnp.float32),
        mesh=mesh,
        compiler_params=pltpu.CompilerParams(use_tc_tiling_on_sc=False),
        scratch_types=[
            pltpu.VMEM((NCH, K), jnp.int32),
            pltpu.VMEM((K,), jnp.float32),
            pltpu.VMEM((SL,), jnp.float32),
            pltpu.VMEM_SHARED((NPAD,), jnp.float32),
        ],
    )
    scatter_kernel = pl.kernel(
        _scatter_body,
        out_type=jax.ShapeDtypeStruct((NC * NPAD, C), jnp.float32),
        mesh=mesh,
        compiler_params=pltpu.CompilerParams(use_tc_tiling_on_sc=False),
        scratch_types=[
            pltpu.VMEM((NCH, K), jnp.int32),
            pltpu.VMEM((NCH, K), jnp.int32),
            pltpu.VMEM((K, C), jnp.float32),
            pltpu.VMEM((SL, C), jnp.float32),
            pltpu.VMEM_SHARED((NPAD, C), jnp.float32),
            pltpu.VMEM_SHARED((NPAD, C), jnp.float32),
        ],
    )
    return deg_kernel, scatter_kernel


def kernel(x, edge_index, batch, W_gcn, b_gcn, W_fc1, b_fc1, W_fc2, b_fc2):
    del batch
    rowp = edge_index[0].reshape(NW, NCH, K)
    colp = edge_index[1].reshape(NW, NCH, K)
    zeros1 = jnp.zeros((NPAD,), jnp.float32)
    zeros8 = jnp.zeros((NPAD, C), jnp.float32)

    deg_kernel, scatter_kernel = _sc_kernels()
    degp = deg_kernel(colp, zeros1)
    d0 = jnp.pad(degp[:N].reshape(N // BR, BR),
                 ((0, 0), (0, DB - BR))).reshape(-1)
    d1 = jnp.pad(degp[NPAD:NPAD + N].reshape(N // BR, BR),
                 ((0, 0), (0, DB - BR))).reshape(-1)
    g = _g_call(x, W_gcn, d0, d1)
    sp = scatter_kernel(g, rowp, colp, zeros8)
    sp = sp.reshape(NC, NPAD, C)
    out = _head_call(sp, g, d0, d1, b_gcn.reshape(1, C), W_fc1,
                     b_fc1.reshape(1, 1), W_fc2, b_fc2.reshape(1, NCLS))
    return out


# speedup vs baseline: 57.0371x; 1.2507x over previous
import functools

import jax
import jax.numpy as jnp
import numpy as np
from jax import lax
from jax.experimental import pallas as pl
from jax.experimental.pallas import tpu as pltpu
from jax.experimental.pallas import tpu_sc as plsc

N = 20000
NN = 10000
E = 320000
F = 128
C = 8
NCLS = 10
SLOPE = 0.01

NC = 2
NS = 16
NW = NC * NS
K = 80
NCH = E // (NW * K)
NPAD = 20096
SL = NPAD // NS

def _deg_body(col_hbm, zeros1_hbm, deg_out, colv, onesv, stage, acc):
    cid = lax.axis_index("c")
    sid = lax.axis_index("s")
    wid = cid * NS + sid
    for i in range(K // 16):
        onesv[pl.ds(i * 16, 16)] = jnp.ones((16,), jnp.float32)
    pltpu.sync_copy(zeros1_hbm.at[pl.ds(sid * SL, SL)], stage)
    pltpu.sync_copy(stage, acc.at[pl.ds(sid * SL, SL)])
    pltpu.sync_copy(col_hbm.at[wid], colv)
    plsc.subcore_barrier()

    def body(j, carry):
        pltpu.sync_copy(onesv, acc.at[colv.at[j]], add=True)
        return carry

    lax.fori_loop(0, NCH, body, 0)
    plsc.subcore_barrier()
    pltpu.sync_copy(acc.at[pl.ds(sid * SL, SL)], stage)
    pltpu.sync_copy(stage, deg_out.at[pl.ds(cid * NPAD + sid * SL, SL)])


def _scatter_body(g_hbm, row_hbm, col_hbm, zeros8_hbm, s_out,
                  rowv, colv, rbuf, stage, g_sh, acc):
    cid = lax.axis_index("c")
    sid = lax.axis_index("s")
    wid = cid * NS + sid
    pltpu.sync_copy(g_hbm.at[pl.ds(sid * SL, SL)], stage)
    pltpu.sync_copy(stage, g_sh.at[pl.ds(sid * SL, SL)])

    @pl.when(cid == 0)
    def _():
        pltpu.sync_copy(stage, acc.at[pl.ds(sid * SL, SL)])

    @pl.when(cid != 0)
    def _():
        pltpu.sync_copy(zeros8_hbm.at[pl.ds(sid * SL, SL)], stage)
        pltpu.sync_copy(stage, acc.at[pl.ds(sid * SL, SL)])

    pltpu.sync_copy(row_hbm.at[wid], rowv)
    pltpu.sync_copy(col_hbm.at[wid], colv)
    plsc.subcore_barrier()

    def body(j, carry):
        pltpu.sync_copy(g_sh.at[rowv.at[j]], rbuf)
        pltpu.sync_copy(rbuf, acc.at[colv.at[j]], add=True)
        return carry

    lax.fori_loop(0, NCH, body, 0)
    plsc.subcore_barrier()
    pltpu.sync_copy(acc.at[pl.ds(sid * SL, SL)], stage)
    pltpu.sync_copy(stage, s_out.at[pl.ds(cid * NPAD + sid * SL, SL)])


BX = 2000


DB = 2048


def _g_body(x_ref, w_ref, d0_ref, d1_ref, g_ref):
    h = jnp.dot(x_ref[...], w_ref[...], preferred_element_type=jnp.float32)
    deg = d0_ref[...] + d1_ref[...] + 1.0
    g_ref[...] = h * jnp.reshape(lax.rsqrt(deg), (DB, 1))[:BX]


_g_call = pl.pallas_call(
    _g_body,
    grid=(N // BX,),
    in_specs=[
        pl.BlockSpec((BX, F), lambda i: (i, 0)),
        pl.BlockSpec((F, C), lambda i: (0, 0)),
        pl.BlockSpec((DB,), lambda i: (i,)),
        pl.BlockSpec((DB,), lambda i: (i,)),
    ],
    out_specs=pl.BlockSpec((BX, C), lambda i: (i, 0)),
    out_shape=jax.ShapeDtypeStruct((NPAD, C), jnp.float32),
)


QW = NPAD * C // 128
_EXPAND = np.zeros((16, 128), np.float32)
for _k in range(16):
    _EXPAND[_k, _k * 8:(_k + 1) * 8] = 1.0
_EXPAND = jnp.asarray(_EXPAND)
_QSEL = np.zeros((16 * NCLS, NCLS), np.float32)
for _j in range(NCLS):
    _QSEL[_j * 16:(_j + 1) * 16, _j] = 1.0
_QSEL = jnp.asarray(_QSEL)
_G0 = NN * C // 128
_G1 = 2 * _G0


def _leaky(v):
    return jnp.where(v >= 0, v, SLOPE * v)


def _head_body(sp_ref, d16_ref, ex_ref, bt_ref, p_ref, b1_ref, w2l_ref,
               qs_ref, b2_ref, out_ref):
    d16 = d16_ref[0] + d16_ref[1] + 1.0
    dinv8 = jnp.dot(lax.rsqrt(d16), ex_ref[...],
                    preferred_element_type=jnp.float32)
    s = (sp_ref[0] + sp_ref[1]) * dinv8 + bt_ref[...]
    a = jnp.dot(_leaky(s), p_ref[...],
                preferred_element_type=jnp.float32) + b1_ref[0, 0]
    y = _leaky(a)
    yrep = jnp.concatenate([y] * NCLS, axis=1)
    prod = yrep * w2l_ref[...]
    rid = lax.broadcasted_iota(jnp.int32, (QW, 1), 0)
    s0 = jnp.sum(jnp.where(rid < _G0, prod, 0.0), axis=0, keepdims=True)
    s1 = jnp.sum(jnp.where((rid >= _G0) & (rid < _G1), prod, 0.0),
                 axis=0, keepdims=True)
    zrow = jnp.concatenate([s0, s1], axis=0)
    z = jnp.dot(zrow, qs_ref[...],
                preferred_element_type=jnp.float32) + b2_ref[...]
    m = jnp.max(z, axis=1, keepdims=True)
    lse = jnp.log(jnp.sum(jnp.exp(z - m), axis=1, keepdims=True)) + m
    out_ref[...] = z - lse


_head_call = pl.pallas_call(
    _head_body,
    out_shape=jax.ShapeDtypeStruct((2, NCLS), jnp.float32),
)


@functools.cache
def _sc_kernels():
    mesh = plsc.VectorSubcoreMesh(core_axis_name="c", subcore_axis_name="s",
                                  num_cores=NC, num_subcores=NS)
    deg_kernel = pl.kernel(
        _deg_body,
        out_type=jax.ShapeDtypeStruct((NC * NPAD,), jnp.float32),
        mesh=mesh,
        compiler_params=pltpu.CompilerParams(use_tc_tiling_on_sc=False),
        scratch_types=[
            pltpu.VMEM((NCH, K), jnp.int32),
            pltpu.VMEM((K,), jnp.float32),
            pltpu.VMEM((SL,), jnp.float32),
            pltpu.VMEM_SHARED((NPAD,), jnp.float32),
        ],
    )
    scatter_kernel = pl.kernel(
        _scatter_body,
        out_type=jax.ShapeDtypeStruct((NC * NPAD, C), jnp.float32),
        mesh=mesh,
        compiler_params=pltpu.CompilerParams(use_tc_tiling_on_sc=False),
        scratch_types=[
            pltpu.VMEM((NCH, K), jnp.int32),
            pltpu.VMEM((NCH, K), jnp.int32),
            pltpu.VMEM((K, C), jnp.float32),
            pltpu.VMEM((SL, C), jnp.float32),
            pltpu.VMEM_SHARED((NPAD, C), jnp.float32),
            pltpu.VMEM_SHARED((NPAD, C), jnp.float32),
        ],
    )
    return deg_kernel, scatter_kernel


def kernel(x, edge_index, batch, W_gcn, b_gcn, W_fc1, b_fc1, W_fc2, b_fc2):
    del batch
    rowp = edge_index[0].reshape(NW, NCH, K)
    colp = edge_index[1].reshape(NW, NCH, K)
    zeros1 = jnp.zeros((NPAD,), jnp.float32)
    zeros8 = jnp.zeros((NPAD, C), jnp.float32)

    deg_kernel, scatter_kernel = _sc_kernels()
    degp = deg_kernel(colp, zeros1)
    d0 = jnp.pad(degp[:N].reshape(N // BX, BX),
                 ((0, 0), (0, DB - BX))).reshape(-1)
    d1 = jnp.pad(degp[NPAD:NPAD + N].reshape(N // BX, BX),
                 ((0, 0), (0, DB - BX))).reshape(-1)
    g = _g_call(x, W_gcn, d0, d1)
    sp = scatter_kernel(g, rowp, colp, zeros8)
    sp_wide = sp.reshape(NC, QW, 128)
    d16p = degp.reshape(NC, QW, 16)
    bt = jnp.tile(b_gcn, 16).reshape(1, 128)
    pmat = jnp.kron(jnp.eye(16, dtype=jnp.float32), W_fc1)
    w2r = W_fc2.reshape(_G0, 16, NCLS)
    w2l = jnp.concatenate(
        [w2r, w2r, jnp.zeros((QW - _G1, 16, NCLS), jnp.float32)],
        axis=0).transpose(0, 2, 1).reshape(QW, 16 * NCLS)
    out = _head_call(sp_wide, d16p, _EXPAND, bt, pmat,
                     b_fc1.reshape(1, 1), w2l, _QSEL, b_fc2.reshape(1, NCLS))
    return out


# speedup vs baseline: 70.6617x; 1.2389x over previous
import functools

import jax
import jax.numpy as jnp
import numpy as np
from jax import lax
from jax.experimental import pallas as pl
from jax.experimental.pallas import tpu as pltpu
from jax.experimental.pallas import tpu_sc as plsc

N = 20000
NN = 10000
E = 320000
F = 128
C = 8
NCLS = 10
SLOPE = 0.01

NC = 2
NS = 16
NW = NC * NS
K = 80
NCH = E // (NW * K)
NPAD = 20096
SL = NPAD // NS

def _deg_body(col_hbm, zeros1_hbm, deg_out, colv, onesv, stage, acc, sem):
    cid = lax.axis_index("c")
    sid = lax.axis_index("s")
    wid = cid * NS + sid
    for i in range(K // 16):
        onesv[pl.ds(i * 16, 16)] = jnp.ones((16,), jnp.float32)
    pltpu.sync_copy(zeros1_hbm.at[pl.ds(sid * SL, SL)], stage)
    pltpu.sync_copy(stage, acc.at[pl.ds(sid * SL, SL)])
    pltpu.sync_copy(col_hbm.at[wid], colv)
    plsc.subcore_barrier()

    def fire(j, carry):
        pltpu.async_copy(onesv, acc.at[colv.at[j]], sem, add=True)
        return carry

    lax.fori_loop(0, NCH, fire, 0)

    def drain(j, carry):
        pltpu.make_async_copy(onesv, acc.at[colv.at[j]], sem).wait()
        return carry

    lax.fori_loop(0, NCH, drain, 0)
    plsc.subcore_barrier()
    pltpu.sync_copy(acc.at[pl.ds(sid * SL, SL)], stage)
    pltpu.sync_copy(stage, deg_out.at[pl.ds(cid * NPAD + sid * SL, SL)])


def _scatter_body(g_hbm, row_hbm, col_hbm, zeros8_hbm, s_out,
                  rowv, colv, rbuf, stage, g_sh, acc, sem):
    cid = lax.axis_index("c")
    sid = lax.axis_index("s")
    wid = cid * NS + sid
    pltpu.sync_copy(g_hbm.at[pl.ds(sid * SL, SL)], stage)
    pltpu.sync_copy(stage, g_sh.at[pl.ds(sid * SL, SL)])

    @pl.when(cid == 0)
    def _():
        pltpu.sync_copy(stage, acc.at[pl.ds(sid * SL, SL)])

    @pl.when(cid != 0)
    def _():
        pltpu.sync_copy(zeros8_hbm.at[pl.ds(sid * SL, SL)], stage)
        pltpu.sync_copy(stage, acc.at[pl.ds(sid * SL, SL)])

    pltpu.sync_copy(row_hbm.at[wid], rowv)
    pltpu.sync_copy(col_hbm.at[wid], colv)
    plsc.subcore_barrier()

    def fire(j, carry):
        pltpu.async_copy(g_sh.at[rowv.at[j]], rbuf.at[j], sem)
        return carry

    lax.fori_loop(0, NCH, fire, 0)

    def drain(j, carry):
        pltpu.make_async_copy(g_sh.at[rowv.at[j]], rbuf.at[j], sem).wait()
        pltpu.sync_copy(rbuf.at[j], acc.at[colv.at[j]], add=True)
        return carry

    lax.fori_loop(0, NCH, drain, 0)
    plsc.subcore_barrier()
    pltpu.sync_copy(acc.at[pl.ds(sid * SL, SL)], stage)
    pltpu.sync_copy(stage, s_out.at[pl.ds(cid * NPAD + sid * SL, SL)])


BX = 2000


DB = 2048


def _g_body(x_ref, w_ref, d0_ref, d1_ref, g_ref):
    h = jnp.dot(x_ref[...], w_ref[...], preferred_element_type=jnp.float32)
    deg = d0_ref[...] + d1_ref[...] + 1.0
    g_ref[...] = h * jnp.reshape(lax.rsqrt(deg), (DB, 1))[:BX]


_g_call = pl.pallas_call(
    _g_body,
    grid=(N // BX,),
    in_specs=[
        pl.BlockSpec((BX, F), lambda i: (i, 0)),
        pl.BlockSpec((F, C), lambda i: (0, 0)),
        pl.BlockSpec((DB,), lambda i: (i,)),
        pl.BlockSpec((DB,), lambda i: (i,)),
    ],
    out_specs=pl.BlockSpec((BX, C), lambda i: (i, 0)),
    out_shape=jax.ShapeDtypeStruct((NPAD, C), jnp.float32),
)


QW = NPAD * C // 128
_EXPAND = np.zeros((16, 128), np.float32)
for _k in range(16):
    _EXPAND[_k, _k * 8:(_k + 1) * 8] = 1.0
_QSEL = np.zeros((16 * NCLS, NCLS), np.float32)
for _j in range(NCLS):
    _QSEL[_j * 16:(_j + 1) * 16, _j] = 1.0
_G0 = NN * C // 128
_G1 = 2 * _G0


def _leaky(v):
    return jnp.where(v >= 0, v, SLOPE * v)


def _head_body(sp_ref, d16_ref, ex_ref, bt_ref, p_ref, b1_ref, w2l_ref,
               qs_ref, b2_ref, out_ref):
    d16 = d16_ref[0] + d16_ref[1] + 1.0
    dinv8 = jnp.dot(lax.rsqrt(d16), ex_ref[...],
                    preferred_element_type=jnp.float32)
    s = (sp_ref[0] + sp_ref[1]) * dinv8 + bt_ref[...]
    a = jnp.dot(_leaky(s), p_ref[...],
                preferred_element_type=jnp.float32) + b1_ref[0, 0]
    y = _leaky(a)
    yrep = jnp.concatenate([y] * NCLS, axis=1)
    prod = yrep * w2l_ref[...]
    rid = lax.broadcasted_iota(jnp.int32, (QW, 1), 0)
    s0 = jnp.sum(jnp.where(rid < _G0, prod, 0.0), axis=0, keepdims=True)
    s1 = jnp.sum(jnp.where((rid >= _G0) & (rid < _G1), prod, 0.0),
                 axis=0, keepdims=True)
    zrow = jnp.concatenate([s0, s1], axis=0)
    z = jnp.dot(zrow, qs_ref[...],
                preferred_element_type=jnp.float32) + b2_ref[...]
    m = jnp.max(z, axis=1, keepdims=True)
    lse = jnp.log(jnp.sum(jnp.exp(z - m), axis=1, keepdims=True)) + m
    out_ref[...] = z - lse


_head_call = pl.pallas_call(
    _head_body,
    out_shape=jax.ShapeDtypeStruct((2, NCLS), jnp.float32),
)


@functools.cache
def _sc_kernels():
    mesh = plsc.VectorSubcoreMesh(core_axis_name="c", subcore_axis_name="s",
                                  num_cores=NC, num_subcores=NS)
    deg_kernel = pl.kernel(
        _deg_body,
        out_type=jax.ShapeDtypeStruct((NC * NPAD,), jnp.float32),
        mesh=mesh,
        compiler_params=pltpu.CompilerParams(use_tc_tiling_on_sc=False),
        scratch_types=[
            pltpu.VMEM((NCH, K), jnp.int32),
            pltpu.VMEM((K,), jnp.float32),
            pltpu.VMEM((SL,), jnp.float32),
            pltpu.VMEM_SHARED((NPAD,), jnp.float32),
            pltpu.SemaphoreType.DMA,
        ],
    )
    scatter_kernel = pl.kernel(
        _scatter_body,
        out_type=jax.ShapeDtypeStruct((NC * NPAD, C), jnp.float32),
        mesh=mesh,
        compiler_params=pltpu.CompilerParams(use_tc_tiling_on_sc=False),
        scratch_types=[
            pltpu.VMEM((NCH, K), jnp.int32),
            pltpu.VMEM((NCH, K), jnp.int32),
            pltpu.VMEM((NCH, K, C), jnp.float32),
            pltpu.VMEM((SL, C), jnp.float32),
            pltpu.VMEM_SHARED((NPAD, C), jnp.float32),
            pltpu.VMEM_SHARED((NPAD, C), jnp.float32),
            pltpu.SemaphoreType.DMA,
        ],
    )
    return deg_kernel, scatter_kernel


def kernel(x, edge_index, batch, W_gcn, b_gcn, W_fc1, b_fc1, W_fc2, b_fc2):
    del batch
    rowp = edge_index[0].reshape(NW, NCH, K)
    colp = edge_index[1].reshape(NW, NCH, K)
    zeros1 = jnp.zeros((NPAD,), jnp.float32)
    zeros8 = jnp.zeros((NPAD, C), jnp.float32)

    deg_kernel, scatter_kernel = _sc_kernels()
    degp = deg_kernel(colp, zeros1)
    d0 = jnp.pad(degp[:N].reshape(N // BX, BX),
                 ((0, 0), (0, DB - BX))).reshape(-1)
    d1 = jnp.pad(degp[NPAD:NPAD + N].reshape(N // BX, BX),
                 ((0, 0), (0, DB - BX))).reshape(-1)
    g = _g_call(x, W_gcn, d0, d1)
    sp = scatter_kernel(g, rowp, colp, zeros8)
    sp_wide = sp.reshape(NC, QW, 128)
    d16p = degp.reshape(NC, QW, 16)
    bt = jnp.tile(b_gcn, 16).reshape(1, 128)
    pmat = jnp.kron(jnp.eye(16, dtype=jnp.float32), W_fc1)
    w2r = W_fc2.reshape(_G0, 16, NCLS)
    w2l = jnp.concatenate(
        [w2r, w2r, jnp.zeros((QW - _G1, 16, NCLS), jnp.float32)],
        axis=0).transpose(0, 2, 1).reshape(QW, 16 * NCLS)
    out = _head_call(sp_wide, d16p, _EXPAND, bt, pmat,
                     b_fc1.reshape(1, 1), w2l, _QSEL, b_fc2.reshape(1, NCLS))
    return out


# speedup vs baseline: 77.0123x; 1.0899x over previous
import functools

import jax
import jax.numpy as jnp
import numpy as np
from jax import lax
from jax.experimental import pallas as pl
from jax.experimental.pallas import tpu as pltpu
from jax.experimental.pallas import tpu_sc as plsc

N = 20000
NN = 10000
E = 320000
F = 128
C = 8
NCLS = 10
SLOPE = 0.01

NC = 2
NS = 16
NW = NC * NS
K = 80
NCH = E // (NW * K)
NPAD = 20096
SL = NPAD // NS

def _deg_body(col_hbm, zeros1_hbm, deg_out, colv, onesv, stage, acc, sem):
    cid = lax.axis_index("c")
    sid = lax.axis_index("s")
    wid = cid * NS + sid
    for i in range(K // 16):
        onesv[pl.ds(i * 16, 16)] = jnp.ones((16,), jnp.float32)
    pltpu.sync_copy(zeros1_hbm.at[pl.ds(sid * SL, SL)], stage)
    pltpu.sync_copy(stage, acc.at[pl.ds(sid * SL, SL)])
    pltpu.sync_copy(col_hbm.at[wid], colv)
    plsc.subcore_barrier()

    def fire(j, carry):
        pltpu.async_copy(onesv, acc.at[colv.at[j]], sem, add=True)
        return carry

    lax.fori_loop(0, NCH, fire, 0)

    def drain(j, carry):
        pltpu.make_async_copy(onesv, acc.at[colv.at[j]], sem).wait()
        return carry

    lax.fori_loop(0, NCH, drain, 0)
    plsc.subcore_barrier()
    pltpu.sync_copy(acc.at[pl.ds(sid * SL, SL)], stage)
    pltpu.sync_copy(stage, deg_out.at[pl.ds(cid * NPAD + sid * SL, SL)])


def _scatter_body(g_hbm, row_hbm, col_hbm, zeros8_hbm, s_out,
                  rowv, colv, rbuf, stage, g_sh, acc, sem, sem2):
    cid = lax.axis_index("c")
    sid = lax.axis_index("s")
    wid = cid * NS + sid
    pltpu.sync_copy(g_hbm.at[pl.ds(sid * SL, SL)], stage)
    pltpu.sync_copy(stage, g_sh.at[pl.ds(sid * SL, SL)])

    @pl.when(cid == 0)
    def _():
        pltpu.sync_copy(stage, acc.at[pl.ds(sid * SL, SL)])

    @pl.when(cid != 0)
    def _():
        pltpu.sync_copy(zeros8_hbm.at[pl.ds(sid * SL, SL)], stage)
        pltpu.sync_copy(stage, acc.at[pl.ds(sid * SL, SL)])

    pltpu.sync_copy(row_hbm.at[wid], rowv)
    pltpu.sync_copy(col_hbm.at[wid], colv)
    plsc.subcore_barrier()

    def fire(j, carry):
        pltpu.async_copy(g_sh.at[rowv.at[j]], rbuf.at[j], sem)
        return carry

    lax.fori_loop(0, NCH, fire, 0)

    def relay(j, carry):
        pltpu.make_async_copy(g_sh.at[rowv.at[j]], rbuf.at[j], sem).wait()
        pltpu.async_copy(rbuf.at[j], acc.at[colv.at[j]], sem2, add=True)
        return carry

    lax.fori_loop(0, NCH, relay, 0)

    def drain(j, carry):
        pltpu.make_async_copy(rbuf.at[j], acc.at[colv.at[j]], sem2).wait()
        return carry

    lax.fori_loop(0, NCH, drain, 0)
    plsc.subcore_barrier()
    pltpu.sync_copy(acc.at[pl.ds(sid * SL, SL)], stage)
    pltpu.sync_copy(stage, s_out.at[pl.ds(cid * NPAD + sid * SL, SL)])


BX = 2000


DB = 2048


def _g_body(x_ref, w_ref, d0_ref, d1_ref, g_ref):
    h = jnp.dot(x_ref[...], w_ref[...], preferred_element_type=jnp.float32)
    deg = d0_ref[...] + d1_ref[...] + 1.0
    g_ref[...] = h * jnp.reshape(lax.rsqrt(deg), (DB, 1))[:BX]


_g_call = pl.pallas_call(
    _g_body,
    grid=(N // BX,),
    in_specs=[
        pl.BlockSpec((BX, F), lambda i: (i, 0)),
        pl.BlockSpec((F, C), lambda i: (0, 0)),
        pl.BlockSpec((DB,), lambda i: (i,)),
        pl.BlockSpec((DB,), lambda i: (i,)),
    ],
    out_specs=pl.BlockSpec((BX, C), lambda i: (i, 0)),
    out_shape=jax.ShapeDtypeStruct((NPAD, C), jnp.float32),
)


QW = NPAD * C // 128
_EXPAND = np.zeros((16, 128), np.float32)
for _k in range(16):
    _EXPAND[_k, _k * 8:(_k + 1) * 8] = 1.0
_QSEL = np.zeros((16 * NCLS, NCLS), np.float32)
for _j in range(NCLS):
    _QSEL[_j * 16:(_j + 1) * 16, _j] = 1.0
_G0 = NN * C // 128
_G1 = 2 * _G0


def _leaky(v):
    return jnp.where(v >= 0, v, SLOPE * v)


def _head_body(sp_ref, d16_ref, ex_ref, bt_ref, p_ref, b1_ref, w2l_ref,
               qs_ref, b2_ref, out_ref):
    d16 = d16_ref[0] + d16_ref[1] + 1.0
    dinv8 = jnp.dot(lax.rsqrt(d16), ex_ref[...],
                    preferred_element_type=jnp.float32)
    s = (sp_ref[0] + sp_ref[1]) * dinv8 + bt_ref[...]
    a = jnp.dot(_leaky(s), p_ref[...],
                preferred_element_type=jnp.float32) + b1_ref[0, 0]
    y = _leaky(a)
    yrep = jnp.concatenate([y] * NCLS, axis=1)
    prod = yrep * w2l_ref[...]
    rid = lax.broadcasted_iota(jnp.int32, (QW, 1), 0)
    s0 = jnp.sum(jnp.where(rid < _G0, prod, 0.0), axis=0, keepdims=True)
    s1 = jnp.sum(jnp.where((rid >= _G0) & (rid < _G1), prod, 0.0),
                 axis=0, keepdims=True)
    zrow = jnp.concatenate([s0, s1], axis=0)
    z = jnp.dot(zrow, qs_ref[...],
                preferred_element_type=jnp.float32) + b2_ref[...]
    m = jnp.max(z, axis=1, keepdims=True)
    lse = jnp.log(jnp.sum(jnp.exp(z - m), axis=1, keepdims=True)) + m
    out_ref[...] = z - lse


_head_call = pl.pallas_call(
    _head_body,
    out_shape=jax.ShapeDtypeStruct((2, NCLS), jnp.float32),
)


@functools.cache
def _sc_kernels():
    mesh = plsc.VectorSubcoreMesh(core_axis_name="c", subcore_axis_name="s",
                                  num_cores=NC, num_subcores=NS)
    deg_kernel = pl.kernel(
        _deg_body,
        out_type=jax.ShapeDtypeStruct((NC * NPAD,), jnp.float32),
        mesh=mesh,
        compiler_params=pltpu.CompilerParams(use_tc_tiling_on_sc=False),
        scratch_types=[
            pltpu.VMEM((NCH, K), jnp.int32),
            pltpu.VMEM((K,), jnp.float32),
            pltpu.VMEM((SL,), jnp.float32),
            pltpu.VMEM_SHARED((NPAD,), jnp.float32),
            pltpu.SemaphoreType.DMA,
        ],
    )
    scatter_kernel = pl.kernel(
        _scatter_body,
        out_type=jax.ShapeDtypeStruct((NC * NPAD, C), jnp.float32),
        mesh=mesh,
        compiler_params=pltpu.CompilerParams(use_tc_tiling_on_sc=False),
        scratch_types=[
            pltpu.VMEM((NCH, K), jnp.int32),
            pltpu.VMEM((NCH, K), jnp.int32),
            pltpu.VMEM((NCH, K, C), jnp.float32),
            pltpu.VMEM((SL, C), jnp.float32),
            pltpu.VMEM_SHARED((NPAD, C), jnp.float32),
            pltpu.VMEM_SHARED((NPAD, C), jnp.float32),
            pltpu.SemaphoreType.DMA,
            pltpu.SemaphoreType.DMA,
        ],
    )
    return deg_kernel, scatter_kernel


def kernel(x, edge_index, batch, W_gcn, b_gcn, W_fc1, b_fc1, W_fc2, b_fc2):
    del batch
    rowp = edge_index[0].reshape(NW, NCH, K)
    colp = edge_index[1].reshape(NW, NCH, K)
    zeros1 = jnp.zeros((NPAD,), jnp.float32)
    zeros8 = jnp.zeros((NPAD, C), jnp.float32)

    deg_kernel, scatter_kernel = _sc_kernels()
    degp = deg_kernel(colp, zeros1)
    d0 = jnp.pad(degp[:N].reshape(N // BX, BX),
                 ((0, 0), (0, DB - BX))).reshape(-1)
    d1 = jnp.pad(degp[NPAD:NPAD + N].reshape(N // BX, BX),
                 ((0, 0), (0, DB - BX))).reshape(-1)
    g = _g_call(x, W_gcn, d0, d1)
    sp = scatter_kernel(g, rowp, colp, zeros8)
    sp_wide = sp.reshape(NC, QW, 128)
    d16p = degp.reshape(NC, QW, 16)
    bt = jnp.tile(b_gcn, 16).reshape(1, 128)
    pmat = jnp.kron(jnp.eye(16, dtype=jnp.float32), W_fc1)
    w2r = W_fc2.reshape(_G0, 16, NCLS)
    w2l = jnp.concatenate(
        [w2r, w2r, jnp.zeros((QW - _G1, 16, NCLS), jnp.float32)],
        axis=0).transpose(0, 2, 1).reshape(QW, 16 * NCLS)
    out = _head_call(sp_wide, d16p, _EXPAND, bt, pmat,
                     b_fc1.reshape(1, 1), w2l, _QSEL, b_fc2.reshape(1, NCLS))
    return out


# speedup vs baseline: 79.0910x; 1.0270x over previous
import functools

import jax
import jax.numpy as jnp
import numpy as np
from jax import lax
from jax.experimental import pallas as pl
from jax.experimental.pallas import tpu as pltpu
from jax.experimental.pallas import tpu_sc as plsc

N = 20000
NN = 10000
E = 320000
F = 128
C = 8
NCLS = 10
SLOPE = 0.01

NC = 2
NS = 16
NW = NC * NS
K = 80
NCH = E // (NW * K)
NPAD = 20096
SL = NPAD // NS

def _deg_body(col_hbm, zeros1_hbm, deg_out, colv, onesv, stage, acc, sem):
    cid = lax.axis_index("c")
    sid = lax.axis_index("s")
    wid = cid * NS + sid
    for i in range(K // 16):
        onesv[pl.ds(i * 16, 16)] = jnp.ones((16,), jnp.float32)
    pltpu.sync_copy(zeros1_hbm.at[pl.ds(sid * SL, SL)], stage)
    pltpu.sync_copy(stage, acc.at[pl.ds(sid * SL, SL)])
    pltpu.sync_copy(col_hbm.at[wid], colv)
    plsc.subcore_barrier()

    def fire(j, carry):
        pltpu.async_copy(onesv, acc.at[colv.at[j]], sem, add=True)
        return carry

    lax.fori_loop(0, NCH, fire, 0)

    def drain(j, carry):
        pltpu.make_async_copy(onesv, acc.at[colv.at[j]], sem).wait()
        return carry

    lax.fori_loop(0, NCH, drain, 0)
    plsc.subcore_barrier()
    pltpu.sync_copy(acc.at[pl.ds(sid * SL, SL)], stage)
    pltpu.sync_copy(stage, deg_out.at[pl.ds(cid * NPAD + sid * SL, SL)])


def _scatter_body(g_hbm, row_hbm, col_hbm, zeros8_hbm, s_out,
                  rowv, colv, rbuf, stage, g_sh, acc, sem, sem2):
    cid = lax.axis_index("c")
    sid = lax.axis_index("s")
    wid = cid * NS + sid
    pltpu.sync_copy(g_hbm.at[pl.ds(sid * SL, SL)], stage)
    pltpu.sync_copy(stage, g_sh.at[pl.ds(sid * SL, SL)])

    @pl.when(cid == 0)
    def _():
        pltpu.sync_copy(stage, acc.at[pl.ds(sid * SL, SL)])

    @pl.when(cid != 0)
    def _():
        pltpu.sync_copy(zeros8_hbm.at[pl.ds(sid * SL, SL)], stage)
        pltpu.sync_copy(stage, acc.at[pl.ds(sid * SL, SL)])

    pltpu.sync_copy(row_hbm.at[wid], rowv)
    pltpu.sync_copy(col_hbm.at[wid], colv)
    plsc.subcore_barrier()

    def fire(j, carry):
        pltpu.async_copy(g_sh.at[rowv.at[j]], rbuf.at[j], sem)
        return carry

    lax.fori_loop(0, NCH, fire, 0)

    def relay(j, carry):
        pltpu.make_async_copy(g_sh.at[rowv.at[j]], rbuf.at[j], sem).wait()
        pltpu.async_copy(rbuf.at[j], acc.at[colv.at[j]], sem2, add=True)
        return carry

    lax.fori_loop(0, NCH, relay, 0)

    def drain(j, carry):
        pltpu.make_async_copy(rbuf.at[j], acc.at[colv.at[j]], sem2).wait()
        return carry

    lax.fori_loop(0, NCH, drain, 0)
    plsc.subcore_barrier()
    pltpu.sync_copy(acc.at[pl.ds(sid * SL, SL)], stage)
    pltpu.sync_copy(stage, s_out.at[pl.ds(cid * NPAD + sid * SL, SL)])


def _g_body(x_ref, w_ref, d_ref, g_ref):
    hT = lax.dot_general(w_ref[...], x_ref[...], (((0,), (1,)), ((), ())),
                         preferred_element_type=jnp.float32)
    degv = d_ref[...]
    deg = degv[:NPAD] + degv[NPAD:] + 1.0
    dinv = jnp.reshape(lax.rsqrt(deg), (1, NPAD))
    g_ref[...] = jnp.pad(hT, ((0, 0), (0, NPAD - N))) * dinv


_g_call = pl.pallas_call(
    _g_body,
    out_shape=jax.ShapeDtypeStruct((C, NPAD), jnp.float32),
)


QW = NPAD * C // 128
_EXPAND = np.zeros((16, 128), np.float32)
for _k in range(16):
    _EXPAND[_k, _k * 8:(_k + 1) * 8] = 1.0
_QSEL = np.zeros((16 * NCLS, NCLS), np.float32)
for _j in range(NCLS):
    _QSEL[_j * 16:(_j + 1) * 16, _j] = 1.0
_G0 = NN * C // 128
_G1 = 2 * _G0


def _leaky(v):
    return jnp.where(v >= 0, v, SLOPE * v)


def _head_body(sp_ref, d16_ref, ex_ref, bt_ref, p_ref, b1_ref, w2l_ref,
               qs_ref, b2_ref, out_ref):
    d16 = d16_ref[0] + d16_ref[1] + 1.0
    dinv8 = jnp.dot(lax.rsqrt(d16), ex_ref[...],
                    preferred_element_type=jnp.float32)
    s = (sp_ref[0] + sp_ref[1]) * dinv8 + bt_ref[...]
    a = jnp.dot(_leaky(s), p_ref[...],
                preferred_element_type=jnp.float32) + b1_ref[0, 0]
    y = _leaky(a)
    yrep = jnp.concatenate([y] * NCLS, axis=1)
    prod = yrep * w2l_ref[...]
    rid = lax.broadcasted_iota(jnp.int32, (QW, 1), 0)
    s0 = jnp.sum(jnp.where(rid < _G0, prod, 0.0), axis=0, keepdims=True)
    s1 = jnp.sum(jnp.where((rid >= _G0) & (rid < _G1), prod, 0.0),
                 axis=0, keepdims=True)
    zrow = jnp.concatenate([s0, s1], axis=0)
    z = jnp.dot(zrow, qs_ref[...],
                preferred_element_type=jnp.float32) + b2_ref[...]
    m = jnp.max(z, axis=1, keepdims=True)
    lse = jnp.log(jnp.sum(jnp.exp(z - m), axis=1, keepdims=True)) + m
    out_ref[...] = z - lse


_head_call = pl.pallas_call(
    _head_body,
    out_shape=jax.ShapeDtypeStruct((2, NCLS), jnp.float32),
)


@functools.cache
def _sc_kernels():
    mesh = plsc.VectorSubcoreMesh(core_axis_name="c", subcore_axis_name="s",
                                  num_cores=NC, num_subcores=NS)
    deg_kernel = pl.kernel(
        _deg_body,
        out_type=jax.ShapeDtypeStruct((NC * NPAD,), jnp.float32),
        mesh=mesh,
        compiler_params=pltpu.CompilerParams(use_tc_tiling_on_sc=False),
        scratch_types=[
            pltpu.VMEM((NCH, K), jnp.int32),
            pltpu.VMEM((K,), jnp.float32),
            pltpu.VMEM((SL,), jnp.float32),
            pltpu.VMEM_SHARED((NPAD,), jnp.float32),
            pltpu.SemaphoreType.DMA,
        ],
    )
    scatter_kernel = pl.kernel(
        _scatter_body,
        out_type=jax.ShapeDtypeStruct((NC * NPAD, C), jnp.float32),
        mesh=mesh,
        compiler_params=pltpu.CompilerParams(use_tc_tiling_on_sc=False),
        scratch_types=[
            pltpu.VMEM((NCH, K), jnp.int32),
            pltpu.VMEM((NCH, K), jnp.int32),
            pltpu.VMEM((NCH, K, C), jnp.float32),
            pltpu.VMEM((SL, C), jnp.float32),
            pltpu.VMEM_SHARED((NPAD, C), jnp.float32),
            pltpu.VMEM_SHARED((NPAD, C), jnp.float32),
            pltpu.SemaphoreType.DMA,
            pltpu.SemaphoreType.DMA,
        ],
    )
    return deg_kernel, scatter_kernel


def kernel(x, edge_index, batch, W_gcn, b_gcn, W_fc1, b_fc1, W_fc2, b_fc2):
    del batch
    rowp = edge_index[0].reshape(NW, NCH, K)
    colp = edge_index[1].reshape(NW, NCH, K)
    zeros1 = jnp.zeros((NPAD,), jnp.float32)
    zeros8 = jnp.zeros((NPAD, C), jnp.float32)

    deg_kernel, scatter_kernel = _sc_kernels()
    degp = deg_kernel(colp, zeros1)
    g = _g_call(x, W_gcn, degp).T
    sp = scatter_kernel(g, rowp, colp, zeros8)
    sp_wide = sp.reshape(NC, QW, 128)
    d16p = degp.reshape(NC, QW, 16)
    bt = jnp.tile(b_gcn, 16).reshape(1, 128)
    pmat = jnp.kron(jnp.eye(16, dtype=jnp.float32), W_fc1)
    w2r = W_fc2.reshape(_G0, 16, NCLS)
    w2l = jnp.concatenate(
        [w2r, w2r, jnp.zeros((QW - _G1, 16, NCLS), jnp.float32)],
        axis=0).transpose(0, 2, 1).reshape(QW, 16 * NCLS)
    out = _head_call(sp_wide, d16p, _EXPAND, bt, pmat,
                     b_fc1.reshape(1, 1), w2l, _QSEL, b_fc2.reshape(1, NCLS))
    return out
